# baseline (speedup 1.0000x reference)
import functools

import jax
import jax.numpy as jnp
from jax import lax
from jax.experimental import pallas as pl
from jax.experimental.pallas import tpu as pltpu
from jax.experimental.pallas import tpu_sc as plsc

B, N, S = 16, 1, 32
RMS, ROPE = 512, 64
HALF = ROPE // 2
HID = RMS + ROPE
L = 4096
EPS = 1e-5

NC, NS = 2, 16
ROWS_PER_WORKER = B * S // (NC * NS)


def _compute_body(idx_smem, x_ref, e_ref, o_ref, c1_ref, c2_ref, s1_ref,
                  s2_ref, idxf_ref, idxc_ref, g_ref, kc_in_ref,
                  k_ref, v_ref, vs_ref, kc_out_ref, ks_scr, dma_sem):
    b = pl.program_id(0)

    x = x_ref[0]
    ms = jnp.mean(x * x, axis=-1, keepdims=True)
    v = x * lax.rsqrt(ms + EPS) * g_ref[...]

    e = e_ref[0]
    o = o_ref[0]
    kh1 = e * c1_ref[0] - o * s1_ref[0]
    kh2 = o * c2_ref[0] + e * s2_ref[0]
    k = jnp.concatenate([kh1, kh2], axis=-1)

    k_ref[0] = k
    v_ref[0] = v

    row = jnp.broadcast_to(idxf_ref[0], (S, S))
    col = jnp.broadcast_to(idxc_ref[0], (S, S))
    eq = col == row
    tpos = lax.broadcasted_iota(jnp.int32, (S, S), 1)
    last = jnp.max(jnp.where(eq, tpos, -1), axis=1, keepdims=True)
    p = (tpos == last).astype(jnp.float32)

    vs_ref[0] = lax.dot_general(p, v, (((1,), (0,)), ((), ())),
                                preferred_element_type=jnp.float32,
                                precision=lax.Precision.HIGHEST)

    ks_scr[...] = lax.dot_general(p, k, (((1,), (0,)), ((), ())),
                                  preferred_element_type=jnp.float32,
                                  precision=lax.Precision.HIGHEST)
    copies = []
    for s in range(S):
        r = idx_smem[b, s]
        cp = pltpu.make_async_copy(
            ks_scr.at[pl.ds(s, 1)], kc_out_ref.at[b, pl.ds(r, 1)], dma_sem)
        cp.start()
        copies.append(cp)
    for cp in copies:
        cp.wait()


@functools.cache
def _compute():
  return pl.pallas_call(
    _compute_body,
    grid=(B,),
    interpret=False,
    in_specs=[
        pl.BlockSpec(memory_space=pltpu.SMEM),
        pl.BlockSpec((1, S, RMS), lambda b: (b, 0, 0)),
        pl.BlockSpec((1, S, HALF), lambda b: (b, 0, 0)),
        pl.BlockSpec((1, S, HALF), lambda b: (b, 0, 0)),
        pl.BlockSpec((1, S, HALF), lambda b: (b, 0, 0)),
        pl.BlockSpec((1, S, HALF), lambda b: (b, 0, 0)),
        pl.BlockSpec((1, S, HALF), lambda b: (b, 0, 0)),
        pl.BlockSpec((1, S, HALF), lambda b: (b, 0, 0)),
        pl.BlockSpec((1, 1, S), lambda b: (b, 0, 0)),
        pl.BlockSpec((1, S, 1), lambda b: (b, 0, 0)),
        pl.BlockSpec((1, RMS), lambda b: (0, 0)),
        pl.BlockSpec(memory_space=pl.ANY),
    ],
    out_specs=[
        pl.BlockSpec((1, S, ROPE), lambda b: (b, 0, 0)),
        pl.BlockSpec((1, S, RMS), lambda b: (b, 0, 0)),
        pl.BlockSpec((1, S, RMS), lambda b: (b, 0, 0)),
        pl.BlockSpec(memory_space=pl.ANY),
    ],
    out_shape=[
        jax.ShapeDtypeStruct((B, S, ROPE), jnp.float32),
        jax.ShapeDtypeStruct((B, S, RMS), jnp.float32),
        jax.ShapeDtypeStruct((B, S, RMS), jnp.float32),
        jax.ShapeDtypeStruct((B, L, ROPE), jnp.float32),
    ],
    scratch_shapes=[
        pltpu.VMEM((S, ROPE), jnp.float32),
        pltpu.SemaphoreType.DMA,
    ],
    input_output_aliases={11: 3},
  )


def _scatter_body(v_hbm, idx_hbm, cc_ref, idx_v, gidx_v, vbuf, sem_v):
    c = lax.axis_index("c")
    s = lax.axis_index("s")
    b = s
    base = ROWS_PER_WORKER * c
    pltpu.sync_copy(idx_hbm.at[b, pl.ds(base, ROWS_PER_WORKER)], idx_v)
    pltpu.sync_copy(v_hbm.at[b, pl.ds(base, ROWS_PER_WORKER)], vbuf)
    gidx_v[...] = idx_v[...] + b * L
    pltpu.async_copy(vbuf, cc_ref.at[gidx_v], sem_v).wait()


@functools.cache
def _sc_scatter():
    return pl.kernel(
        _scatter_body,
        out_type=(),
        interpret=False,
        mesh=plsc.VectorSubcoreMesh(
            core_axis_name="c", subcore_axis_name="s",
            num_cores=NC, num_subcores=NS),
        scratch_types=[
            pltpu.VMEM((ROWS_PER_WORKER,), jnp.int32),
            pltpu.VMEM((ROWS_PER_WORKER,), jnp.int32),
            pltpu.VMEM((ROWS_PER_WORKER, RMS), jnp.float32),
            pltpu.SemaphoreType.DMA,
        ],
    )


def kernel(kv, gamma, cos, sin, index, k_cache, ckv_cache):
    kvs = kv.reshape(B, S, HID)
    x_rms = kvs[..., :RMS]
    rope = kvs[..., RMS:]
    rope_e = rope[..., 0::2]
    rope_o = rope[..., 1::2]
    cs = cos.reshape(B, S, ROPE)
    sn = sin.reshape(B, S, ROPE)
    idxf = index.astype(jnp.float32)

    k_vals, v_vals, v_scat, k_cache_out = _compute()(
        index, x_rms, rope_e, rope_o,
        cs[..., :HALF], cs[..., HALF:], sn[..., :HALF], sn[..., HALF:],
        idxf.reshape(B, 1, S), idxf.reshape(B, S, 1),
        gamma.reshape(1, RMS), k_cache.reshape(B, L, ROPE))

    ckv_ref = jax.new_ref(ckv_cache.reshape(B * L, RMS))
    _sc_scatter()(v_scat, index, ckv_ref)

    return (k_vals.reshape(B, N, S, ROPE),
            v_vals.reshape(B, N, S, RMS),
            k_cache_out.reshape(B, N, L, ROPE),
            ckv_ref[...].reshape(B, N, L, RMS))
